# Initial kernel scaffold; baseline (speedup 1.0000x reference)
import jax
import jax.numpy as jnp
from jax.experimental import pallas as pl


def kernel(boxes, scores):
    raise NotImplementedError("write your pallas kernel here")


# speedup vs baseline: 19.8779x; 19.8779x over previous
import functools

import jax
import jax.numpy as jnp
from jax import lax
from jax.experimental import pallas as pl
from jax.experimental.pallas import tpu as pltpu
from jax.experimental.pallas import tpu_sc as plsc

N = 20000
K = 1024
K_REAL = 1000
OUT_K = 256
SCORE_THRESH = 0.05
NMS_THRESH = 0.9

ROWS = 32
COLS = 1024
TOT = ROWS * COLS

BLK = 8


def _topk_body(s_ref, key_ref, idx_ref):
    x = s_ref[...]
    key = jnp.where(x > SCORE_THRESH, x, jnp.where(x >= 0.0, 0.0, -1.0))
    r = lax.broadcasted_iota(jnp.int32, (ROWS, COLS), 0)
    c = lax.broadcasted_iota(jnp.int32, (ROWS, COLS), 1)
    flat = r * COLS + c
    idx = flat

    def cx(key, idx, k, d):
        if d < COLS:
            kp_m = jnp.roll(key, -d, axis=1)
            kp_p = jnp.roll(key, d, axis=1)
            ip_m = jnp.roll(idx, -d, axis=1)
            ip_p = jnp.roll(idx, d, axis=1)
        else:
            m = d // COLS
            kp_m = jnp.roll(key, -m, axis=0)
            kp_p = jnp.roll(key, m, axis=0)
            ip_m = jnp.roll(idx, -m, axis=0)
            ip_p = jnp.roll(idx, m, axis=0)
        upper = (flat & d) == 0
        key_p = jnp.where(upper, kp_m, kp_p)
        idx_p = jnp.where(upper, ip_m, ip_p)
        self_first = (key > key_p) | ((key == key_p) & (idx < idx_p))
        descblk = (flat & k) == 0
        take_self = self_first == (upper == descblk)
        return (jnp.where(take_self, key, key_p),
                jnp.where(take_self, idx, idx_p))

    k = 2
    while k <= TOT:
        d = k // 2
        while d >= 1:
            key, idx = cx(key, idx, k, d)
            d //= 2
        k *= 2

    key_ref[...] = key[0:1, :]
    idx_ref[...] = idx[0:1, :]


_topk_call = pl.pallas_call(
    _topk_body,
    out_shape=(jax.ShapeDtypeStruct((1, COLS), jnp.float32),
               jax.ShapeDtypeStruct((1, COLS), jnp.int32)),
)


_NC, _NS = 2, 16
_NW = _NC * _NS
_BPW = K // _NW

@functools.lru_cache(maxsize=1)
def _make_gather_boxes():
    mesh = plsc.VectorSubcoreMesh(core_axis_name="c", subcore_axis_name="s",
                                  num_cores=_NC, num_subcores=_NS)

    @functools.partial(
        pl.kernel,
        out_type=jax.ShapeDtypeStruct((K, 16), jnp.float32),
        mesh=mesh,
        scratch_types=[pltpu.VMEM((_BPW,), jnp.int32),
                       pltpu.VMEM((_BPW, 16), jnp.float32),
                       pltpu.SemaphoreType.DMA],
        compiler_params=pltpu.CompilerParams(use_tc_tiling_on_sc=False),
    )
    def _gather_boxes(boxes_hbm, idx_hbm, out_hbm, idx_v, rows_v, sem):
        wid = lax.axis_index("s") * _NC + lax.axis_index("c")
        base = wid * _BPW
        pltpu.sync_copy(idx_hbm.at[pl.ds(base, _BPW)], idx_v)
        pltpu.async_copy(boxes_hbm.at[idx_v], rows_v, sem).wait()
        pltpu.sync_copy(rows_v, out_hbm.at[pl.ds(base, _BPW)])

    return _gather_boxes


def _nms_body(tb_ref, tbt_ref, ts_ref, out_ref, m_ref, keep_ref):
    x1r = tbt_ref[0:1, :]
    y1r = tbt_ref[1:2, :]
    x2r = tbt_ref[2:3, :]
    y2r = tbt_ref[3:4, :]
    area_r = jnp.maximum(x2r - x1r, 0.0) * jnp.maximum(y2r - y1r, 0.0)
    jglob = lax.broadcasted_iota(jnp.int32, (BLK, COLS), 1)

    def build(i, _):
        rb = tb_ref[pl.ds(i * BLK, BLK), :]
        x1c = rb[:, 0:1]
        y1c = rb[:, 1:2]
        x2c = rb[:, 2:3]
        y2c = rb[:, 3:4]
        area_c = (jnp.maximum(x2c - x1c, 0.0) *
                  jnp.maximum(y2c - y1c, 0.0))
        w = jnp.maximum(jnp.minimum(x2c, x2r) - jnp.maximum(x1c, x1r), 0.0)
        h = jnp.maximum(jnp.minimum(y2c, y2r) - jnp.maximum(y1c, y1r), 0.0)
        inter = w * h
        iou = inter / jnp.maximum(area_c + area_r - inter, 1e-9)
        iglob = i * BLK + lax.broadcasted_iota(jnp.int32, (BLK, COLS), 0)
        m = ((iou > NMS_THRESH) & (jglob > iglob)
             & (iglob < K_REAL) & (jglob < K_REAL))
        m_ref[pl.ds(i * BLK, BLK), :] = m.astype(jnp.float32)
        return 0

    lax.fori_loop(0, COLS // BLK, build, 0)

    keep_ref[...] = jnp.ones((1, COLS), jnp.float32)
    pos = lax.broadcasted_iota(jnp.int32, (1, COLS), 1)

    def step(i, _):
        keep = keep_ref[...]
        ki = jnp.max(jnp.where(pos == i, keep, 0.0))
        row = m_ref[pl.ds(i, 1), :]
        keep_ref[...] = keep * (1.0 - row * ki)
        return 0

    lax.fori_loop(0, K_REAL, step, 0)
    fs = ts_ref[...] * keep_ref[...]
    fs = jnp.where(pos < K_REAL, fs, -1.0)

    def cx6(vals, k, d):
        upper = (pos & d) == 0
        parts = [jnp.where(upper, jnp.roll(v, -d, axis=1),
                           jnp.roll(v, d, axis=1)) for v in vals]
        key, idx = vals[0], vals[1]
        key_p, idx_p = parts[0], parts[1]
        self_first = (key > key_p) | ((key == key_p) & (idx < idx_p))
        descblk = (pos & k) == 0
        take_self = self_first == (upper == descblk)
        return [jnp.where(take_self, v, vp) for v, vp in zip(vals, parts)]

    vals = [fs, pos, x1r, y1r, x2r, y2r]
    k = 2
    while k <= COLS:
        d = k // 2
        while d >= 1:
            vals = cx6(vals, k, d)
            d //= 2
        k *= 2

    out_ref[0:1, :] = lax.slice(vals[2], (0, 0), (1, OUT_K))
    out_ref[1:2, :] = lax.slice(vals[3], (0, 0), (1, OUT_K))
    out_ref[2:3, :] = lax.slice(vals[4], (0, 0), (1, OUT_K))
    out_ref[3:4, :] = lax.slice(vals[5], (0, 0), (1, OUT_K))
    out_ref[4:5, :] = lax.slice(vals[0], (0, 0), (1, OUT_K))


_nms_call = pl.pallas_call(
    _nms_body,
    out_shape=jax.ShapeDtypeStruct((5, OUT_K), jnp.float32),
    scratch_shapes=[pltpu.VMEM((COLS, COLS), jnp.float32),
                    pltpu.VMEM((1, COLS), jnp.float32)],
)


def kernel(boxes, scores):
    s_pad = jnp.concatenate(
        [scores.astype(jnp.float32),
         jnp.full((TOT - N,), -1.0, jnp.float32)]).reshape(ROWS, COLS)
    key_row, idx_row = _topk_call(s_pad)
    top_idx = idx_row.reshape(K)
    boxes16 = jnp.pad(boxes, ((0, 0), (0, 12)))
    top_boxes = _make_gather_boxes()(boxes16, top_idx)[:, :4]
    res = _nms_call(top_boxes, top_boxes.T, key_row)
    return res.T


# speedup vs baseline: 43.8047x; 2.2037x over previous
import functools

import jax
import jax.numpy as jnp
from jax import lax
from jax.experimental import pallas as pl
from jax.experimental.pallas import tpu as pltpu
from jax.experimental.pallas import tpu_sc as plsc

N = 20000
K = 1024
K_REAL = 1000
OUT_K = 256
SCORE_THRESH = 0.05
NMS_THRESH = 0.9

ROWS = 32
COLS = 1024
TOT = ROWS * COLS

BLK = 8


def _topk_body(s_ref, key_ref, idx_ref):
    x = s_ref[...]
    key = jnp.where(x > SCORE_THRESH, x, jnp.where(x >= 0.0, 0.0, -1.0))
    r = lax.broadcasted_iota(jnp.int32, (ROWS, COLS), 0)
    c = lax.broadcasted_iota(jnp.int32, (ROWS, COLS), 1)
    flat = r * COLS + c
    idx = flat

    def cx(key, idx, k, d):
        if d < COLS:
            kp_m = jnp.roll(key, -d, axis=1)
            kp_p = jnp.roll(key, d, axis=1)
            ip_m = jnp.roll(idx, -d, axis=1)
            ip_p = jnp.roll(idx, d, axis=1)
        else:
            m = d // COLS
            kp_m = jnp.roll(key, -m, axis=0)
            kp_p = jnp.roll(key, m, axis=0)
            ip_m = jnp.roll(idx, -m, axis=0)
            ip_p = jnp.roll(idx, m, axis=0)
        upper = (flat & d) == 0
        key_p = jnp.where(upper, kp_m, kp_p)
        idx_p = jnp.where(upper, ip_m, ip_p)
        self_first = (key > key_p) | ((key == key_p) & (idx < idx_p))
        descblk = (flat & k) == 0
        take_self = self_first == (upper == descblk)
        return (jnp.where(take_self, key, key_p),
                jnp.where(take_self, idx, idx_p))

    k = 2
    while k <= TOT:
        d = k // 2
        while d >= 1:
            key, idx = cx(key, idx, k, d)
            d //= 2
        k *= 2

    key_ref[...] = key[0:1, :]
    idx_ref[...] = idx[0:1, :]


_topk_call = pl.pallas_call(
    _topk_body,
    out_shape=(jax.ShapeDtypeStruct((1, COLS), jnp.float32),
               jax.ShapeDtypeStruct((1, COLS), jnp.int32)),
)


_NC, _NS = 2, 16
_NW = _NC * _NS
_BPW = K // _NW

@functools.lru_cache(maxsize=1)
def _make_gather_boxes():
    mesh = plsc.VectorSubcoreMesh(core_axis_name="c", subcore_axis_name="s",
                                  num_cores=_NC, num_subcores=_NS)

    @functools.partial(
        pl.kernel,
        out_type=jax.ShapeDtypeStruct((K, 16), jnp.float32),
        mesh=mesh,
        scratch_types=[pltpu.VMEM((_BPW,), jnp.int32),
                       pltpu.VMEM((_BPW, 16), jnp.float32),
                       pltpu.SemaphoreType.DMA],
        compiler_params=pltpu.CompilerParams(use_tc_tiling_on_sc=False),
    )
    def _gather_boxes(boxes_hbm, idx_hbm, out_hbm, idx_v, rows_v, sem):
        wid = lax.axis_index("s") * _NC + lax.axis_index("c")
        base = wid * _BPW
        pltpu.sync_copy(idx_hbm.at[pl.ds(base, _BPW)], idx_v)
        pltpu.async_copy(boxes_hbm.at[idx_v], rows_v, sem).wait()
        pltpu.sync_copy(rows_v, out_hbm.at[pl.ds(base, _BPW)])

    return _gather_boxes


def _nms_body(tb_ref, tbt_ref, ts_ref, out_ref, m_ref, keep_ref):
    x1r = tbt_ref[0:1, :]
    y1r = tbt_ref[1:2, :]
    x2r = tbt_ref[2:3, :]
    y2r = tbt_ref[3:4, :]
    area_r = jnp.maximum(x2r - x1r, 0.0) * jnp.maximum(y2r - y1r, 0.0)
    jglob = lax.broadcasted_iota(jnp.int32, (BLK, COLS), 1)

    def build(i, _):
        rb = tb_ref[pl.ds(i * BLK, BLK), :]
        x1c = rb[:, 0:1]
        y1c = rb[:, 1:2]
        x2c = rb[:, 2:3]
        y2c = rb[:, 3:4]
        area_c = (jnp.maximum(x2c - x1c, 0.0) *
                  jnp.maximum(y2c - y1c, 0.0))
        w = jnp.maximum(jnp.minimum(x2c, x2r) - jnp.maximum(x1c, x1r), 0.0)
        h = jnp.maximum(jnp.minimum(y2c, y2r) - jnp.maximum(y1c, y1r), 0.0)
        inter = w * h
        iou = inter / jnp.maximum(area_c + area_r - inter, 1e-9)
        iglob = i * BLK + lax.broadcasted_iota(jnp.int32, (BLK, COLS), 0)
        m = ((iou > NMS_THRESH) & (jglob > iglob)
             & (iglob < K_REAL) & (jglob < K_REAL))
        m_ref[pl.ds(i * BLK, BLK), :] = m.astype(jnp.float32)
        return 0

    lax.fori_loop(0, COLS // BLK, build, 0)

    pos = lax.broadcasted_iota(jnp.int32, (1, COLS), 1)

    def sweep(carry):
        keep, _ = carry
        s = jnp.dot(keep, m_ref[...], preferred_element_type=jnp.float32)
        new = (s == 0.0).astype(jnp.float32)
        return new, jnp.any(new != keep)

    keep, _ = lax.while_loop(
        lambda c: c[1], sweep,
        (jnp.ones((1, COLS), jnp.float32), jnp.bool_(True)))
    keep_ref[...] = keep
    fs = ts_ref[...] * keep_ref[...]
    fs = jnp.where(pos < K_REAL, fs, -1.0)

    def cx6(vals, k, d):
        upper = (pos & d) == 0
        parts = [jnp.where(upper, jnp.roll(v, -d, axis=1),
                           jnp.roll(v, d, axis=1)) for v in vals]
        key, idx = vals[0], vals[1]
        key_p, idx_p = parts[0], parts[1]
        self_first = (key > key_p) | ((key == key_p) & (idx < idx_p))
        descblk = (pos & k) == 0
        take_self = self_first == (upper == descblk)
        return [jnp.where(take_self, v, vp) for v, vp in zip(vals, parts)]

    vals = [fs, pos, x1r, y1r, x2r, y2r]
    k = 2
    while k <= COLS:
        d = k // 2
        while d >= 1:
            vals = cx6(vals, k, d)
            d //= 2
        k *= 2

    out_ref[0:1, :] = lax.slice(vals[2], (0, 0), (1, OUT_K))
    out_ref[1:2, :] = lax.slice(vals[3], (0, 0), (1, OUT_K))
    out_ref[2:3, :] = lax.slice(vals[4], (0, 0), (1, OUT_K))
    out_ref[3:4, :] = lax.slice(vals[5], (0, 0), (1, OUT_K))
    out_ref[4:5, :] = lax.slice(vals[0], (0, 0), (1, OUT_K))


_nms_call = pl.pallas_call(
    _nms_body,
    out_shape=jax.ShapeDtypeStruct((5, OUT_K), jnp.float32),
    scratch_shapes=[pltpu.VMEM((COLS, COLS), jnp.float32),
                    pltpu.VMEM((1, COLS), jnp.float32)],
)


def kernel(boxes, scores):
    s_pad = jnp.concatenate(
        [scores.astype(jnp.float32),
         jnp.full((TOT - N,), -1.0, jnp.float32)]).reshape(ROWS, COLS)
    key_row, idx_row = _topk_call(s_pad)
    top_idx = idx_row.reshape(K)
    boxes16 = jnp.pad(boxes, ((0, 0), (0, 12)))
    top_boxes = _make_gather_boxes()(boxes16, top_idx)[:, :4]
    res = _nms_call(top_boxes, top_boxes.T, key_row)
    return res.T
